# baseline (speedup 1.0000x reference)
import functools

import jax
import jax.numpy as jnp
from jax import lax
from jax.experimental import pallas as pl
from jax.experimental.pallas import tpu as pltpu
from jax.experimental.pallas import tpu_sc as plsc


@functools.lru_cache(maxsize=None)
def _make_gather(batch: int, n_rows: int, d_model: int):
    info = plsc.get_sparse_core_info()
    num_cores, num_subcores = info.num_cores, info.num_subcores
    nw = num_cores * num_subcores
    assert batch % nw == 0
    b_per_w = batch // nw
    mesh = plsc.VectorSubcoreMesh(core_axis_name="c", subcore_axis_name="s")

    @functools.partial(
        pl.kernel,
        mesh=mesh,
        out_type=jax.ShapeDtypeStruct((batch, d_model), jnp.float32),
        scratch_types=[
            pltpu.VMEM((b_per_w,), jnp.int32),
            pltpu.VMEM((b_per_w, d_model), jnp.float32),
            pltpu.SemaphoreType.DMA,
        ],
        compiler_params=pltpu.CompilerParams(use_tc_tiling_on_sc=False),
    )
    def gather_kernel(idx_hbm, table_hbm, out_hbm, idx_v, rows_v, sem):
        wid = lax.axis_index("s") * num_cores + lax.axis_index("c")
        base = wid * b_per_w
        pltpu.sync_copy(idx_hbm.at[pl.ds(base, b_per_w)], idx_v)
        pltpu.async_copy(table_hbm.at[idx_v], rows_v, sem).wait()
        pltpu.sync_copy(rows_v, out_hbm.at[pl.ds(base, b_per_w)])

    return gather_kernel


def kernel(x, table):
    batch = x.shape[0]
    n_rows, d_model = table.shape
    idx = x.reshape(batch).astype(jnp.int32)
    out = _make_gather(batch, n_rows, d_model)(idx, table)
    return out.reshape(batch, 1, d_model)


# speedup vs baseline: 3.8763x; 3.8763x over previous
import functools

import jax
import jax.numpy as jnp
from jax import lax
from jax.experimental import pallas as pl
from jax.experimental.pallas import tpu as pltpu
from jax.experimental.pallas import tpu_sc as plsc

_FIRE = 16
_GROUP = 128


@functools.lru_cache(maxsize=None)
def _make_gather(batch: int, n_rows: int, d_model: int):
    info = plsc.get_sparse_core_info()
    num_cores, num_subcores = info.num_cores, info.num_subcores
    nl = info.num_lanes
    nw = num_cores * num_subcores
    b_per_w = batch // nw
    mesh = plsc.VectorSubcoreMesh(core_axis_name="c", subcore_axis_name="s")

    @functools.partial(
        pl.kernel,
        mesh=mesh,
        out_type=jax.ShapeDtypeStruct((d_model, batch), jnp.float32),
        scratch_types=[
            pltpu.VMEM((b_per_w,), jnp.int32),
            pltpu.VMEM((_FIRE, d_model, 128), jnp.float32),
            pltpu.VMEM((d_model, _GROUP), jnp.float32),
            pltpu.SemaphoreType.DMA,
        ],
        compiler_params=pltpu.CompilerParams(
            use_tc_tiling_on_sc=True, needs_layout_passes=False
        ),
    )
    def gather_kernel(idx_hbm, tablet_hbm, out_hbm, idx_v, ring_v, grp_v, sem):
        wid = lax.axis_index("s") * num_cores + lax.axis_index("c")
        base = wid * b_per_w
        pltpu.sync_copy(idx_hbm.at[pl.ds(base, b_per_w)], idx_v)

        dvec0 = lax.iota(jnp.int32, nl)
        dvec1 = dvec0 + nl

        def col_dma(i, b):
            j = pl.multiple_of((i // 128) * 128, 128)
            return pltpu.make_async_copy(
                tablet_hbm.at[:, pl.ds(j, 128)],
                ring_v.at[b],
                sem,
            )

        def group(g):
            def chunk(c):
                k0 = g * _GROUP + c * _FIRE
                v16 = idx_v[pl.ds(k0, _FIRE)]
                for b in range(_FIRE):
                    col_dma(v16[b], b).start()
                for b in range(_FIRE):
                    col_dma(v16[b], b).wait()
                    l = v16[b] % 128
                    lvec = jnp.full((nl,), l, jnp.int32)
                    kvec = jnp.full((nl,), c * _FIRE + b, jnp.int32)
                    bvec = jnp.full((nl,), b, jnp.int32)
                    lo = plsc.load_gather(ring_v, [bvec, dvec0, lvec])
                    hi = plsc.load_gather(ring_v, [bvec, dvec1, lvec])
                    plsc.store_scatter(grp_v, [dvec0, kvec], lo)
                    plsc.store_scatter(grp_v, [dvec1, kvec], hi)

            pl.loop(0, _GROUP // _FIRE)(chunk)
            pltpu.sync_copy(
                grp_v,
                out_hbm.at[:, pl.ds(pl.multiple_of(base + g * _GROUP, 128), _GROUP)],
            )

        pl.loop(0, b_per_w // _GROUP)(group)

    return gather_kernel


def kernel(x, table):
    batch = x.shape[0]
    n_rows, d_model = table.shape
    idx = x.reshape(batch).astype(jnp.int32)
    out_t = _make_gather(batch, n_rows, d_model)(idx, table.T)
    return out_t.T.reshape(batch, 1, d_model)


# speedup vs baseline: 4.7517x; 1.2259x over previous
import functools

import jax
import jax.numpy as jnp
from jax import lax
from jax.experimental import pallas as pl
from jax.experimental.pallas import tpu as pltpu
from jax.experimental.pallas import tpu_sc as plsc

_FIRE = 16
_GROUP = 128


@functools.lru_cache(maxsize=None)
def _make_gather(batch: int, n_rows: int, d_model: int):
    info = plsc.get_sparse_core_info()
    num_cores, num_subcores = info.num_cores, info.num_subcores
    nl = info.num_lanes
    nw = num_cores * num_subcores
    b_per_w = batch // nw
    n_chunks = b_per_w // _FIRE
    chunks_per_group = _GROUP // _FIRE
    mesh = plsc.VectorSubcoreMesh(core_axis_name="c", subcore_axis_name="s")

    @functools.partial(
        pl.kernel,
        mesh=mesh,
        out_type=jax.ShapeDtypeStruct((d_model, batch), jnp.float32),
        scratch_types=[
            pltpu.VMEM((b_per_w,), jnp.int32),
            pltpu.VMEM((_FIRE, d_model, 128), jnp.float32),
            pltpu.VMEM((d_model, _GROUP), jnp.float32),
            pltpu.SemaphoreType.DMA,
        ],
        compiler_params=pltpu.CompilerParams(
            use_tc_tiling_on_sc=True, needs_layout_passes=False
        ),
    )
    def gather_kernel(idx_hbm, tablet_hbm, out_hbm, idx_v, ring_v, grp_v, sem):
        wid = lax.axis_index("s") * num_cores + lax.axis_index("c")
        base = wid * b_per_w
        pltpu.sync_copy(idx_hbm.at[pl.ds(base, b_per_w)], idx_v)

        dvec0 = lax.iota(jnp.int32, nl)
        dvec1 = dvec0 + nl

        def col_dma(i, b):
            j = pl.multiple_of((i // 128) * 128, 128)
            return pltpu.make_async_copy(
                tablet_hbm.at[:, pl.ds(j, 128)],
                ring_v.at[b],
                sem,
            )

        def consume(c, b, v_cur):
            col_dma(v_cur[b], b).wait()
            l = v_cur[b] % 128
            lvec = jnp.full((nl,), l, jnp.int32)
            kvec = jnp.full((nl,), (c % chunks_per_group) * _FIRE + b, jnp.int32)
            bvec = jnp.full((nl,), b, jnp.int32)
            lo = plsc.load_gather(ring_v, [bvec, dvec0, lvec])
            hi = plsc.load_gather(ring_v, [bvec, dvec1, lvec])
            plsc.store_scatter(grp_v, [dvec0, kvec], lo)
            plsc.store_scatter(grp_v, [dvec1, kvec], hi)

        def flush(c):
            g = c // chunks_per_group
            pltpu.sync_copy(
                grp_v,
                out_hbm.at[:, pl.ds(pl.multiple_of(base + g * _GROUP, 128), _GROUP)],
            )

        v0 = idx_v[pl.ds(0, _FIRE)]
        for b in range(_FIRE):
            col_dma(v0[b], b).start()

        def chunk(c):
            v_cur = idx_v[pl.ds(c * _FIRE, _FIRE)]
            v_nxt = idx_v[pl.ds((c + 1) * _FIRE, _FIRE)]
            for b in range(_FIRE):
                consume(c, b, v_cur)
                col_dma(v_nxt[b], b).start()

            @pl.when(c % chunks_per_group == chunks_per_group - 1)
            def _():
                flush(c)

        pl.loop(0, n_chunks - 1)(chunk)

        c_last = n_chunks - 1
        v_last = idx_v[pl.ds(c_last * _FIRE, _FIRE)]
        for b in range(_FIRE):
            consume(c_last, b, v_last)
        flush(c_last)

    return gather_kernel


def kernel(x, table):
    batch = x.shape[0]
    n_rows, d_model = table.shape
    idx = x.reshape(batch).astype(jnp.int32)
    out_t = _make_gather(batch, n_rows, d_model)(idx, table.T)
    return out_t.T.reshape(batch, 1, d_model)
